# baseline (speedup 1.0000x reference)
import functools

import jax
import jax.numpy as jnp
from jax import lax
from jax.experimental import pallas as pl
from jax.experimental.pallas import tpu as pltpu
from jax.experimental.pallas import tpu_sc as plsc

N_WORKERS = 32
LANES = 16
T_CHUNK = 32
RSQRT_MAGIC = 0x5F3759DF


def _lane_sum16(x):
    lanes = lax.iota(jnp.int32, 16)
    for off in (1, 2, 4, 8):
        perm = lax.bitwise_xor(lanes, jnp.full((16,), off, jnp.int32))
        x = x + x.at[perm].get(mode="promise_in_bounds")
    return x


def _rsqrt16(x):
    i = plsc.bitcast(x, jnp.int32)
    i = jnp.full((16,), RSQRT_MAGIC, jnp.int32) - lax.shift_right_logical(i, 1)
    y = plsc.bitcast(i, jnp.float32)
    for _ in range(3):
        y = y * (1.5 - 0.5 * x * y * y)
    return y


def kernel(field_tables, position_table, ln_scale, ln_bias, input_ids):
    n_fields, vocab, hidden = field_tables.shape
    batch, seq, _ = input_ids.shape
    n_tok = batch * seq
    assert hidden == 64 and n_fields == 26
    assert n_tok % (N_WORKERS * T_CHUNK) == 0
    per_w = n_tok // N_WORKERS
    n_chunks = per_w // T_CHUNK
    assert n_chunks % 2 == 0
    n_pairs = n_chunks // 2
    n_idx = T_CHUNK * n_fields
    n_groups = n_idx // 64

    flat_tables = field_tables.reshape(n_fields * vocab, hidden)
    offsets = jnp.arange(n_fields, dtype=jnp.int32) * vocab
    ids1d = (input_ids + offsets[None, None, :]).reshape(-1)
    pos1d = (position_table * jnp.float32(1.0)).reshape(-1)

    hv = hidden // LANES

    buf = lambda shp, dt: pltpu.VMEM(shp, dt)

    @functools.partial(
        pl.kernel,
        out_type=jax.ShapeDtypeStruct((n_tok * hidden,), jnp.float32),
        mesh=plsc.VectorSubcoreMesh(core_axis_name="c", subcore_axis_name="s"),
        compiler_params=pltpu.CompilerParams(
            needs_layout_passes=False, use_tc_tiling_on_sc=False),
        scratch_types=[
            buf((2 * n_idx,), jnp.int32), buf((2 * n_idx,), jnp.int32),
            buf((n_idx, hidden), jnp.float32),
            buf((n_idx, hidden), jnp.float32),
            buf((T_CHUNK * hidden,), jnp.float32),
            buf((T_CHUNK * hidden,), jnp.float32),
            buf((seq * hidden,), jnp.float32),
            buf((hidden,), jnp.float32),
            buf((hidden,), jnp.float32),
            pltpu.SemaphoreType.DMA,
            pltpu.SemaphoreType.DMA,
            pltpu.SemaphoreType.DMA,
        ],
    )
    def emb_kernel(table_hbm, ids_hbm, pos_hbm, scale_hbm, bias_hbm,
                   out_hbm,
                   pids0, pids1, rows_v0, rows_v1,
                   out_v0, out_v1, pos_v, scale_v, bias_v,
                   sem0, sem1, sem_ids):
        wid = lax.axis_index("c") * 16 + lax.axis_index("s")

        pltpu.sync_copy(pos_hbm.at[pl.ds(0, seq * hidden)], pos_v)
        pltpu.sync_copy(scale_hbm, scale_v)
        pltpu.sync_copy(bias_hbm, bias_v)

        tok_w0 = wid * per_w
        pbufs = (pids0, pids1)
        rbufs = ((rows_v0, out_v0, sem0), (rows_v1, out_v1, sem1))

        def ids_copy(p, pk):
            id0 = pl.multiple_of((tok_w0 + 2 * p * T_CHUNK) * n_fields, 8)
            return pltpu.make_async_copy(
                ids_hbm.at[pl.ds(id0, 2 * n_idx)], pbufs[pk], sem_ids)

        def fire(half, pk, rk):
            rows_v, _, sem = rbufs[rk]
            for g in range(n_groups):
                pltpu.async_copy(
                    table_hbm.at[
                        pbufs[pk].at[pl.ds(half * n_idx + 64 * g, 64)]],
                    rows_v.at[pl.ds(64 * g, 64)],
                    sem,
                )

        def compute(c, rk):
            rows_v, out_v, sem = rbufs[rk]
            tok0 = tok_w0 + c * T_CHUNK
            for g in range(n_groups):
                pltpu.make_async_copy(
                    table_hbm.at[pbufs[0].at[pl.ds(64 * g, 64)]],
                    rows_v.at[pl.ds(64 * g, 64)],
                    sem,
                ).wait()

            @plsc.parallel_loop(0, T_CHUNK, 1, unroll=2)
            def tok_body(i):
                s = lax.rem(tok0 + i, seq)
                base = i * n_fields
                accs = [pos_v[pl.ds(s * hidden + 16 * j, 16)]
                        for j in range(hv)]
                for f in range(n_fields):
                    for j in range(hv):
                        accs[j] = accs[j] + rows_v[base + f, pl.ds(16 * j, 16)]
                tot = accs[0] + accs[1] + accs[2] + accs[3]
                sqt = (accs[0] * accs[0] + accs[1] * accs[1]
                       + accs[2] * accs[2] + accs[3] * accs[3])
                mean = _lane_sum16(tot) * (1.0 / hidden)
                ex2 = _lane_sum16(sqt) * (1.0 / hidden)
                var = ex2 - mean * mean
                rstd = _rsqrt16(var + 1e-12)
                for j in range(hv):
                    val = ((accs[j] - mean) * rstd
                           * scale_v[pl.ds(16 * j, 16)]
                           + bias_v[pl.ds(16 * j, 16)])
                    out_v[pl.ds(i * hidden + 16 * j, 16)] = val

            pltpu.sync_copy(
                out_v, out_hbm.at[pl.ds(tok0 * hidden, T_CHUNK * hidden)])

        ids_copy(0, 0).start()
        ids_copy(0, 0).wait()
        fire(0, 0, 0)

        def pair_body(p, carry):
            pk = lax.rem(p, 2)
            c = 2 * p

            @pl.when(pk == 0)
            def _():
                fire(1, 0, 1)
                compute(c, 0)

                @pl.when(p + 1 < n_pairs)
                def _():
                    ids_copy(p + 1, 1).start()
                compute(c + 1, 1)

                @pl.when(p + 1 < n_pairs)
                def _():
                    ids_copy(p + 1, 1).wait()
                    fire(0, 1, 0)

            @pl.when(pk == 1)
            def _():
                fire(1, 1, 1)
                compute(c, 0)

                @pl.when(p + 1 < n_pairs)
                def _():
                    ids_copy(p + 1, 0).start()
                compute(c + 1, 1)

                @pl.when(p + 1 < n_pairs)
                def _():
                    ids_copy(p + 1, 0).wait()
                    fire(0, 0, 0)

            return carry

        lax.fori_loop(0, n_pairs, pair_body, 0)

    out = emb_kernel(flat_tables, ids1d, pos1d, ln_scale, ln_bias)
    return out.reshape(batch, seq, hidden)


# speedup vs baseline: 1.1767x; 1.1767x over previous
import functools

import jax
import jax.numpy as jnp
from jax import lax
from jax.experimental import pallas as pl
from jax.experimental.pallas import tpu as pltpu
from jax.experimental.pallas import tpu_sc as plsc

N_WORKERS = 32
LANES = 16
T_CHUNK = 32
RSQRT_MAGIC = 0x5F3759DF


def _lane_sum16(x):
    lanes = lax.iota(jnp.int32, 16)
    for off in (1, 2, 4, 8):
        perm = lax.bitwise_xor(lanes, jnp.full((16,), off, jnp.int32))
        x = x + x.at[perm].get(mode="promise_in_bounds")
    return x


def _rsqrt16(x):
    i = plsc.bitcast(x, jnp.int32)
    i = jnp.full((16,), RSQRT_MAGIC, jnp.int32) - lax.shift_right_logical(i, 1)
    y = plsc.bitcast(i, jnp.float32)
    for _ in range(3):
        y = y * (1.5 - 0.5 * x * y * y)
    return y


def kernel(field_tables, position_table, ln_scale, ln_bias, input_ids):
    n_fields, vocab, hidden = field_tables.shape
    batch, seq, _ = input_ids.shape
    n_tok = batch * seq
    assert hidden == 64 and n_fields == 26
    assert n_tok % (N_WORKERS * T_CHUNK) == 0
    per_w = n_tok // N_WORKERS
    n_chunks = per_w // T_CHUNK
    assert n_chunks % 2 == 0
    n_pairs = n_chunks // 2
    n_idx = T_CHUNK * n_fields
    n_groups = n_idx // 64

    flat_tables = field_tables.reshape(n_fields * vocab, hidden)
    offsets = jnp.arange(n_fields, dtype=jnp.int32) * vocab
    ids1d = (input_ids + offsets[None, None, :]).reshape(-1)
    pos1d = (position_table * jnp.float32(1.0)).reshape(-1)

    hv = hidden // LANES

    buf = lambda shp, dt: pltpu.VMEM(shp, dt)

    @functools.partial(
        pl.kernel,
        out_type=jax.ShapeDtypeStruct((n_tok * hidden,), jnp.float32),
        mesh=plsc.VectorSubcoreMesh(core_axis_name="c", subcore_axis_name="s"),
        compiler_params=pltpu.CompilerParams(
            needs_layout_passes=False, use_tc_tiling_on_sc=False),
        scratch_types=[
            buf((2 * n_idx,), jnp.int32), buf((2 * n_idx,), jnp.int32),
            buf((n_idx, hidden), jnp.float32),
            buf((n_idx, hidden), jnp.float32),
            buf((T_CHUNK * hidden,), jnp.float32),
            buf((T_CHUNK * hidden,), jnp.float32),
            buf((seq * hidden,), jnp.float32),
            buf((hidden,), jnp.float32),
            buf((hidden,), jnp.float32),
            pltpu.SemaphoreType.DMA,
            pltpu.SemaphoreType.DMA,
            pltpu.SemaphoreType.DMA,
        ],
    )
    def emb_kernel(table_hbm, ids_hbm, pos_hbm, scale_hbm, bias_hbm,
                   out_hbm,
                   pids0, pids1, rows_v0, rows_v1,
                   out_v0, out_v1, pos_v, scale_v, bias_v,
                   sem0, sem1, sem_ids):
        wid = lax.axis_index("c") * 16 + lax.axis_index("s")

        pltpu.sync_copy(pos_hbm.at[pl.ds(0, seq * hidden)], pos_v)
        pltpu.sync_copy(scale_hbm, scale_v)
        pltpu.sync_copy(bias_hbm, bias_v)

        tok_w0 = wid * per_w
        pbufs = (pids0, pids1)
        rbufs = ((rows_v0, out_v0, sem0), (rows_v1, out_v1, sem1))

        def ids_copy(p, pk):
            id0 = pl.multiple_of((tok_w0 + 2 * p * T_CHUNK) * n_fields, 8)
            return pltpu.make_async_copy(
                ids_hbm.at[pl.ds(id0, 2 * n_idx)], pbufs[pk], sem_ids)

        def fire(half, pk, rk):
            rows_v, _, sem = rbufs[rk]
            for g in range(n_groups):
                pltpu.async_copy(
                    table_hbm.at[
                        pbufs[pk].at[pl.ds(half * n_idx + 64 * g, 64)]],
                    rows_v.at[pl.ds(64 * g, 64)],
                    sem,
                )

        def compute(c, rk):
            rows_v, out_v, sem = rbufs[rk]
            tok0 = tok_w0 + c * T_CHUNK
            for g in range(n_groups):
                pltpu.make_async_copy(
                    table_hbm.at[pbufs[0].at[pl.ds(64 * g, 64)]],
                    rows_v.at[pl.ds(64 * g, 64)],
                    sem,
                ).wait()

            @plsc.parallel_loop(0, T_CHUNK, 1, unroll=2)
            def tok_body(i):
                s = lax.rem(tok0 + i, seq)
                base = i * n_fields
                accs = [pos_v[pl.ds(s * hidden + 16 * j, 16)]
                        for j in range(hv)]
                for f in range(n_fields):
                    for j in range(hv):
                        accs[j] = accs[j] + rows_v[base + f, pl.ds(16 * j, 16)]
                tot = accs[0] + accs[1] + accs[2] + accs[3]
                sqt = (accs[0] * accs[0] + accs[1] * accs[1]
                       + accs[2] * accs[2] + accs[3] * accs[3])
                mean = _lane_sum16(tot) * (1.0 / hidden)
                ex2 = _lane_sum16(sqt) * (1.0 / hidden)
                var = ex2 - mean * mean
                rstd = _rsqrt16(var + 1e-12)
                for j in range(hv):
                    val = ((accs[j] - mean) * rstd
                           * scale_v[pl.ds(16 * j, 16)]
                           + bias_v[pl.ds(16 * j, 16)])
                    out_v[pl.ds(i * hidden + 16 * j, 16)] = val

            pltpu.sync_copy(
                out_v, out_hbm.at[pl.ds(tok0 * hidden, T_CHUNK * hidden)])

        ids_copy(0, 0).start()
        ids_copy(0, 0).wait()
        fire(0, 0, 0)
        ids_copy(1, 1).start()

        n_quads = n_pairs // 2

        def quad_body(q, carry):
            c = 4 * q
            not_last = q + 1 < n_quads

            fire(1, 0, 1)
            compute(c, 0)
            ids_copy(2 * q + 1, 1).wait()
            fire(0, 1, 0)
            compute(c + 1, 1)

            @pl.when(not_last)
            def _():
                ids_copy(2 * q + 2, 0).start()

            fire(1, 1, 1)
            compute(c + 2, 0)

            @pl.when(not_last)
            def _():
                ids_copy(2 * q + 2, 0).wait()
                fire(0, 0, 0)
            compute(c + 3, 1)

            @pl.when(not_last)
            def _():
                ids_copy(2 * q + 3, 1).start()
            return carry

        assert n_pairs % 2 == 0
        lax.fori_loop(0, n_quads, quad_body, 0)

    out = emb_kernel(flat_tables, ids1d, pos1d, ln_scale, ln_bias)
    return out.reshape(batch, seq, hidden)


# speedup vs baseline: 1.1888x; 1.0103x over previous
import functools

import jax
import jax.numpy as jnp
from jax import lax
from jax.experimental import pallas as pl
from jax.experimental.pallas import tpu as pltpu
from jax.experimental.pallas import tpu_sc as plsc

N_WORKERS = 32
LANES = 16
T_CHUNK = 32
RSQRT_MAGIC = 0x5F3759DF


def _lane_sum16(x):
    lanes = lax.iota(jnp.int32, 16)
    for off in (1, 2, 4, 8):
        perm = lax.bitwise_xor(lanes, jnp.full((16,), off, jnp.int32))
        x = x + x.at[perm].get(mode="promise_in_bounds")
    return x


def _rsqrt16(x):
    i = plsc.bitcast(x, jnp.int32)
    i = jnp.full((16,), RSQRT_MAGIC, jnp.int32) - lax.shift_right_logical(i, 1)
    y = plsc.bitcast(i, jnp.float32)
    for _ in range(3):
        y = y * (1.5 - 0.5 * x * y * y)
    return y


def kernel(field_tables, position_table, ln_scale, ln_bias, input_ids):
    n_fields, vocab, hidden = field_tables.shape
    batch, seq, _ = input_ids.shape
    n_tok = batch * seq
    assert hidden == 64 and n_fields == 26
    assert n_tok % (N_WORKERS * T_CHUNK) == 0
    per_w = n_tok // N_WORKERS
    n_chunks = per_w // T_CHUNK
    assert n_chunks % 2 == 0
    n_pairs = n_chunks // 2
    n_idx = T_CHUNK * n_fields
    n_groups = n_idx // 64

    flat_tables = field_tables.reshape(n_fields * vocab, hidden)
    offsets = jnp.arange(n_fields, dtype=jnp.int32) * vocab
    ids1d = (input_ids + offsets[None, None, :]).reshape(-1)
    pos1d = (position_table * jnp.float32(1.0)).reshape(-1)

    hv = hidden // LANES

    buf = lambda shp, dt: pltpu.VMEM(shp, dt)

    @functools.partial(
        pl.kernel,
        out_type=jax.ShapeDtypeStruct((n_tok * hidden,), jnp.float32),
        mesh=plsc.VectorSubcoreMesh(core_axis_name="c", subcore_axis_name="s"),
        compiler_params=pltpu.CompilerParams(
            needs_layout_passes=False, use_tc_tiling_on_sc=False),
        scratch_types=[
            buf((2 * n_idx,), jnp.int32), buf((2 * n_idx,), jnp.int32),
            buf((n_idx, hidden), jnp.float32),
            buf((n_idx, hidden), jnp.float32),
            buf((T_CHUNK * hidden,), jnp.float32),
            buf((T_CHUNK * hidden,), jnp.float32),
            buf((seq * hidden,), jnp.float32),
            buf((hidden,), jnp.float32),
            buf((hidden,), jnp.float32),
            pltpu.SemaphoreType.DMA,
            pltpu.SemaphoreType.DMA,
            pltpu.SemaphoreType.DMA,
            pltpu.SemaphoreType.DMA,
            pltpu.SemaphoreType.DMA,
        ],
    )
    def emb_kernel(table_hbm, ids_hbm, pos_hbm, scale_hbm, bias_hbm,
                   out_hbm,
                   pids0, pids1, rows_v0, rows_v1,
                   out_v0, out_v1, pos_v, scale_v, bias_v,
                   sem0, sem1, sem_ids, osem0, osem1):
        wid = lax.axis_index("c") * 16 + lax.axis_index("s")

        pltpu.sync_copy(pos_hbm.at[pl.ds(0, seq * hidden)], pos_v)
        pltpu.sync_copy(scale_hbm, scale_v)
        pltpu.sync_copy(bias_hbm, bias_v)

        tok_w0 = wid * per_w
        pbufs = (pids0, pids1)
        rbufs = ((rows_v0, out_v0, sem0, osem0),
                 (rows_v1, out_v1, sem1, osem1))

        def ids_copy(p, pk):
            id0 = pl.multiple_of((tok_w0 + 2 * p * T_CHUNK) * n_fields, 8)
            return pltpu.make_async_copy(
                ids_hbm.at[pl.ds(id0, 2 * n_idx)], pbufs[pk], sem_ids)

        def fire(half, pk, rk):
            rows_v, _, sem, _ = rbufs[rk]
            for g in range(n_groups):
                pltpu.async_copy(
                    table_hbm.at[
                        pbufs[pk].at[pl.ds(half * n_idx + 64 * g, 64)]],
                    rows_v.at[pl.ds(64 * g, 64)],
                    sem,
                )

        def compute(c, rk):
            rows_v, out_v, sem, osem = rbufs[rk]
            tok0 = tok_w0 + c * T_CHUNK
            out_cp = pltpu.make_async_copy(
                out_v, out_hbm.at[pl.ds(tok0 * hidden, T_CHUNK * hidden)],
                osem)

            @pl.when(c >= 2)
            def _():
                out_cp.wait()

            for g in range(n_groups):
                pltpu.make_async_copy(
                    table_hbm.at[pbufs[0].at[pl.ds(64 * g, 64)]],
                    rows_v.at[pl.ds(64 * g, 64)],
                    sem,
                ).wait()

            @plsc.parallel_loop(0, T_CHUNK, 1, unroll=2)
            def tok_body(i):
                s = lax.rem(tok0 + i, seq)
                base = i * n_fields
                accs = [pos_v[pl.ds(s * hidden + 16 * j, 16)]
                        for j in range(hv)]
                for f in range(n_fields):
                    for j in range(hv):
                        accs[j] = accs[j] + rows_v[base + f, pl.ds(16 * j, 16)]
                tot = accs[0] + accs[1] + accs[2] + accs[3]
                sqt = (accs[0] * accs[0] + accs[1] * accs[1]
                       + accs[2] * accs[2] + accs[3] * accs[3])
                mean = _lane_sum16(tot) * (1.0 / hidden)
                ex2 = _lane_sum16(sqt) * (1.0 / hidden)
                var = ex2 - mean * mean
                rstd = _rsqrt16(var + 1e-12)
                for j in range(hv):
                    val = ((accs[j] - mean) * rstd
                           * scale_v[pl.ds(16 * j, 16)]
                           + bias_v[pl.ds(16 * j, 16)])
                    out_v[pl.ds(i * hidden + 16 * j, 16)] = val

            out_cp.start()

        ids_copy(0, 0).start()
        ids_copy(0, 0).wait()
        fire(0, 0, 0)
        ids_copy(1, 1).start()

        n_quads = n_pairs // 2

        def quad_body(q, carry):
            c = 4 * q
            not_last = q + 1 < n_quads

            fire(1, 0, 1)
            compute(c, 0)
            ids_copy(2 * q + 1, 1).wait()
            fire(0, 1, 0)
            compute(c + 1, 1)

            @pl.when(not_last)
            def _():
                ids_copy(2 * q + 2, 0).start()

            fire(1, 1, 1)
            compute(c + 2, 0)

            @pl.when(not_last)
            def _():
                ids_copy(2 * q + 2, 0).wait()
                fire(0, 0, 0)
            compute(c + 3, 1)

            @pl.when(not_last)
            def _():
                ids_copy(2 * q + 3, 1).start()
            return carry

        assert n_pairs % 2 == 0
        lax.fori_loop(0, n_quads, quad_body, 0)

        for rk in range(2):
            _, out_v, _, osem = rbufs[rk]
            pltpu.make_async_copy(
                out_v, out_hbm.at[pl.ds(tok_w0 * hidden, T_CHUNK * hidden)],
                osem).wait()

    out = emb_kernel(flat_tables, ids1d, pos1d, ln_scale, ln_bias)
    return out.reshape(batch, seq, hidden)
